# baseline (speedup 1.0000x reference)
import functools

import jax
import jax.numpy as jnp
from jax import lax
from jax.experimental import pallas as pl
from jax.experimental.pallas import tpu as pltpu
from jax.experimental.pallas import tpu_sc as plsc

_N, _D = 8192, 768
_PATHS = 8
_NC, _NS, _L = 2, 16, 16
_NW = _NC * _NS
_RPW = _N // _NW
_C = 128
_NCHUNK = _RPW // _C
_ZR = 32
_NPAD = _RPW // _ZR

_mesh = plsc.VectorSubcoreMesh(core_axis_name="c", subcore_axis_name="s")


@functools.partial(
    pl.kernel,
    mesh=_mesh,
    out_type=[jax.ShapeDtypeStruct((_N, _D), jnp.float32)] * 2,
    scratch_types=[
        pltpu.VMEM((_ZR, _D), jnp.float32),
        pltpu.SemaphoreType.DMA,
        pltpu.SemaphoreType.DMA,
    ],
)
def _sc_dispatch(x_hbm, hit_hbm, pad_hbm, zbuf, hit_sem, pad_sem):
    wid = lax.axis_index("s") * _NC + lax.axis_index("c")
    base = wid * _RPW

    hits = [
        pltpu.async_copy(x_hbm.at[pl.ds(base + k * _C, _C), :],
                         hit_hbm.at[pl.ds(base + k * _C, _C), :], hit_sem)
        for k in range(_NCHUNK)
    ]

    zvec = jnp.zeros((_L,), jnp.float32)

    def _zrow(i, carry):
        def _zcol(j, c):
            zbuf[i, pl.ds(j * _L, _L)] = zvec
            return c
        return lax.fori_loop(0, _D // _L, _zcol, carry)

    lax.fori_loop(0, _ZR, _zrow, 0)

    pads = [
        pltpu.async_copy(zbuf, pad_hbm.at[pl.ds(base + t * _ZR, _ZR), :],
                         pad_sem)
        for t in range(_NPAD)
    ]

    for h in hits:
        h.wait()
    for p in pads:
        p.wait()


def kernel(inputs):
    hit, pad = _sc_dispatch(inputs)
    return (hit,) + (pad,) * (_PATHS - 1)


# speedup vs baseline: 7.3732x; 7.3732x over previous
import functools

import jax
import jax.numpy as jnp
from jax import lax
from jax.experimental import pallas as pl
from jax.experimental.pallas import tpu as pltpu
from jax.experimental.pallas import tpu_sc as plsc

_N, _D = 8192, 768
_PATHS = 8
_NC, _NS, _L = 2, 16, 16
_NW = _NC * _NS
_RPW = _N // _NW
_C = 32
_NCHUNK = _RPW // _C
_NBUF = 4
_ZR = 32
_NPAD = _RPW // _ZR

_mesh = plsc.VectorSubcoreMesh(core_axis_name="c", subcore_axis_name="s")


@functools.partial(
    pl.kernel,
    mesh=_mesh,
    out_type=[jax.ShapeDtypeStruct((_N, _D), jnp.float32)] * 2,
    scratch_types=[
        pltpu.VMEM((_C, _D), jnp.float32),
        pltpu.VMEM((_C, _D), jnp.float32),
        pltpu.VMEM((_C, _D), jnp.float32),
        pltpu.VMEM((_C, _D), jnp.float32),
        pltpu.VMEM((_ZR, _D), jnp.float32),
        pltpu.SemaphoreType.DMA,
        pltpu.SemaphoreType.DMA,
        pltpu.SemaphoreType.DMA,
    ],
)
def _sc_dispatch(x_hbm, hit_hbm, pad_hbm, buf0, buf1, buf2, buf3, zbuf,
                 in_sem, out_sem, pad_sem):
    wid = lax.axis_index("s") * _NC + lax.axis_index("c")
    base = wid * _RPW
    bufs = (buf0, buf1, buf2, buf3)

    def in_cp(k):
        return pltpu.async_copy(x_hbm.at[pl.ds(base + k * _C, _C), :],
                                bufs[k % _NBUF], in_sem)

    def out_cp(k):
        return pltpu.async_copy(bufs[k % _NBUF],
                                hit_hbm.at[pl.ds(base + k * _C, _C), :],
                                out_sem)

    ins = [None] * _NCHUNK
    outs = [None] * _NCHUNK
    ins[0] = in_cp(0)
    ins[1] = in_cp(1)

    zvec = jnp.zeros((_L,), jnp.float32)

    def _zrow(i, carry):
        def _zcol(j, c):
            zbuf[i, pl.ds(j * _L, _L)] = zvec
            return c
        return lax.fori_loop(0, _D // _L, _zcol, carry)

    lax.fori_loop(0, _ZR, _zrow, 0)

    pads = [None] * _NPAD

    def pad_cp(t):
        return pltpu.async_copy(zbuf, pad_hbm.at[pl.ds(base + t * _ZR, _ZR), :],
                                pad_sem)

    for k in range(_NCHUNK):
        ins[k].wait()
        outs[k] = out_cp(k)
        if k + 2 < _NCHUNK:
            if k - 2 >= 0:
                outs[k - 2].wait()
            ins[k + 2] = in_cp(k + 2)
        if k < _NPAD:
            pads[k] = pad_cp(k)
    for k in range(_NCHUNK - 4, _NCHUNK):
        outs[k].wait()
    for p in pads:
        p.wait()


def kernel(inputs):
    hit, pad = _sc_dispatch(inputs)
    return (hit,) + (pad,) * (_PATHS - 1)


# speedup vs baseline: 8.0628x; 1.0935x over previous
import functools

import jax
import jax.numpy as jnp
from jax import lax
from jax.experimental import pallas as pl
from jax.experimental.pallas import tpu as pltpu
from jax.experimental.pallas import tpu_sc as plsc

_N, _D = 8192, 768
_PATHS = 8
_NC, _NS, _L = 2, 16, 16
_NW = _NC * _NS
_RPW = _N // _NW
_ZR = 32
_NPAD = _RPW // _ZR
_BLOCK = 2048

_mesh = plsc.VectorSubcoreMesh(core_axis_name="c", subcore_axis_name="s")


@functools.partial(
    pl.kernel,
    mesh=_mesh,
    out_type=jax.ShapeDtypeStruct((_N, _D), jnp.float32),
    scratch_types=[
        pltpu.VMEM((_ZR, _D), jnp.float32),
        pltpu.SemaphoreType.DMA,
    ],
)
def _sc_pad(pad_hbm, zbuf, pad_sem):
    wid = lax.axis_index("s") * _NC + lax.axis_index("c")
    base = wid * _RPW

    zvec = jnp.zeros((_L,), jnp.float32)

    def _zrow(i, carry):
        def _zcol(j, c):
            zbuf[i, pl.ds(j * _L, _L)] = zvec
            return c
        return lax.fori_loop(0, _D // _L, _zcol, carry)

    lax.fori_loop(0, _ZR, _zrow, 0)

    pads = [
        pltpu.async_copy(zbuf, pad_hbm.at[pl.ds(base + t * _ZR, _ZR), :],
                         pad_sem)
        for t in range(_NPAD)
    ]
    for p in pads:
        p.wait()


def _copy_body(x_ref, hit_ref):
    hit_ref[...] = x_ref[...]


def kernel(inputs):
    n, d = inputs.shape
    pad = _sc_pad()
    hit = pl.pallas_call(
        _copy_body,
        grid=(n // _BLOCK,),
        in_specs=[pl.BlockSpec((_BLOCK, d), lambda i: (i, 0))],
        out_specs=pl.BlockSpec((_BLOCK, d), lambda i: (i, 0)),
        out_shape=jax.ShapeDtypeStruct((n, d), inputs.dtype),
    )(inputs)
    return (hit,) + (pad,) * (_PATHS - 1)
